# baseline (speedup 1.0000x reference)
import functools

import jax
import jax.numpy as jnp
from jax import lax
from jax.experimental import pallas as pl
from jax.experimental.pallas import tpu as pltpu
from jax.experimental.pallas import tpu_sc as plsc

BATCH = 16384
NC, NS, L = 2, 16, 16
NW = NC * NS
B_PER_W = BATCH // NW
CHUNK = 128
NCHUNK = B_PER_W // CHUNK
D = 64
IDXW = 128
BLKR = 8192
NSTR = 4


def _score_kernel(w_ref, *refs):
    xs, os = refs[:NSTR], refs[NSTR:]
    for x_ref, o_ref in zip(xs, os):
        o_ref[...] = lax.dot_general(
            w_ref[...], x_ref[...],
            dimension_numbers=(((1,), (1,)), ((), ())),
            preferred_element_type=jnp.float32)[None]


def _scores(table, w_half):
    rows = table.shape[0]
    grid = (rows + NSTR * BLKR - 1) // (NSTR * BLKR)

    maxb = (rows + BLKR - 1) // BLKR - 1

    def mk_in(k):
        return pl.BlockSpec(
            (BLKR, D),
            lambda i, k=k: (jnp.minimum(NSTR * i + k, maxb), 0))

    outs = pl.pallas_call(
        _score_kernel,
        grid=(grid,),
        in_specs=[pl.BlockSpec((1, D), lambda i: (0, 0))]
        + [mk_in(k) for k in range(NSTR)],
        out_specs=[
            pl.BlockSpec((1, 1, BLKR), lambda i: (i, 0, 0))
            for _ in range(NSTR)
        ],
        out_shape=[
            jax.ShapeDtypeStruct((grid, 1, BLKR), jnp.float32)
            for _ in range(NSTR)
        ],
    )(w_half, *([table] * NSTR))
    merged = jnp.stack([o.reshape(grid, BLKR) for o in outs], axis=1)
    return merged.reshape(grid * NSTR * BLKR // CHUNK, CHUNK)


def _lookup_kernel(urow_hbm, ucol_hbm, irow_hbm, icol_hbm,
                   us_hbm, is_hbm, b_hbm, out_hbm,
                   urow_v, ucol_v, irow_v, icol_v,
                   usc_v, isc_v, b_v, logit_v, usem, isem):
    wid = lax.axis_index("s") * NC + lax.axis_index("c")
    base = wid * NCHUNK

    pltpu.sync_copy(urow_hbm.at[pl.ds(base, NCHUNK)], urow_v)
    pltpu.sync_copy(ucol_hbm.at[pl.ds(base, NCHUNK)], ucol_v)
    pltpu.sync_copy(irow_hbm.at[pl.ds(base, NCHUNK)], irow_v)
    pltpu.sync_copy(icol_hbm.at[pl.ds(base, NCHUNK)], icol_v)
    pltpu.sync_copy(b_hbm, b_v)

    bias = b_v[pl.ds(0, L)]
    rows0 = lax.iota(jnp.int32, L)

    for j in range(NCHUNK):
        cu = pltpu.async_copy(us_hbm.at[urow_v.at[j]], usc_v, usem)
        ci = pltpu.async_copy(is_hbm.at[irow_v.at[j]], isc_v, isem)
        cu.wait()
        ci.wait()

        def group_body(g, _, j=j):
            rows = rows0 + g * L
            uv = plsc.load_gather(usc_v, [rows, ucol_v[j, pl.ds(g * L, L)]])
            iv = plsc.load_gather(isc_v, [rows, icol_v[j, pl.ds(g * L, L)]])
            x = uv + iv + bias
            logit_v[pl.ds(j * CHUNK + g * L, L)] = 1.0 / (1.0 + jnp.exp(-x))
            return 0

        lax.fori_loop(0, CHUNK // L, group_body, 0)

    pltpu.sync_copy(logit_v, out_hbm.at[pl.ds(wid * B_PER_W, B_PER_W)])


@jax.jit
def kernel(batch_user_id, batch_item_id, user_table, item_table, W, b):
    uid = batch_user_id.astype(jnp.int32)
    iid = batch_item_id.astype(jnp.int32)
    urow = (uid >> 7).reshape(BATCH // IDXW, IDXW)
    ucol = (uid & 127).reshape(BATCH // IDXW, IDXW)
    irow = (iid >> 7).reshape(BATCH // IDXW, IDXW)
    icol = (iid & 127).reshape(BATCH // IDXW, IDXW)
    b16 = jnp.broadcast_to(b, (L,))

    us2 = _scores(user_table, W[:D].reshape(1, D))
    is2 = _scores(item_table, W[D:].reshape(1, D))

    run = functools.partial(
        pl.kernel,
        out_type=jax.ShapeDtypeStruct((BATCH,), jnp.float32),
        mesh=plsc.VectorSubcoreMesh(core_axis_name="c", subcore_axis_name="s"),
        compiler_params=pltpu.CompilerParams(needs_layout_passes=False),
        scratch_types=[
            pltpu.VMEM((NCHUNK, IDXW), jnp.int32),
            pltpu.VMEM((NCHUNK, IDXW), jnp.int32),
            pltpu.VMEM((NCHUNK, IDXW), jnp.int32),
            pltpu.VMEM((NCHUNK, IDXW), jnp.int32),
            pltpu.VMEM((CHUNK, CHUNK), jnp.float32),
            pltpu.VMEM((CHUNK, CHUNK), jnp.float32),
            pltpu.VMEM((L,), jnp.float32),
            pltpu.VMEM((B_PER_W,), jnp.float32),
            pltpu.SemaphoreType.DMA,
            pltpu.SemaphoreType.DMA,
        ],
    )(_lookup_kernel)
    out = run(urow, ucol, irow, icol, us2, is2, b16)
    return out.reshape(BATCH, 1)
